# Initial kernel scaffold; baseline (speedup 1.0000x reference)
import functools

import jax
import jax.numpy as jnp
from jax import lax
from jax.experimental import pallas as pl
from jax.experimental.pallas import tpu as pltpu
from jax.experimental.pallas import tpu_sc as plsc

N = 10000
E = 320000
D_IN = 128
H1 = 128
H3 = 64

NC = 2
NS = 16
CH = 128

R_ACC = 10240
ROWS_PER_TILE = R_ACC // NS
EDGES_PER_SC = ((E // NC + NS * CH - 1) // (NS * CH)) * (NS * CH)
EP = EDGES_PER_SC * NC
CHUNKS = EP // CH
CHUNKS_PER_TILE = CHUNKS // (NC * NS)

BLK = 1000
GRID = N // BLK


def _make_deg_kernel():
    mesh = plsc.VectorSubcoreMesh(core_axis_name="c", subcore_axis_name="s")

    @functools.partial(
        pl.kernel,
        out_type=jax.ShapeDtypeStruct((NC, R_ACC, 16), jnp.float32),
        mesh=mesh,
        scratch_types=[
            pltpu.VMEM((CHUNKS_PER_TILE, CH), jnp.int32),
            pltpu.VMEM((CH, 16), jnp.float32),
            pltpu.VMEM((CH, 16), jnp.float32),
            pltpu.VMEM_SHARED((R_ACC, 16), jnp.float32),
        ],
    )
    def deg_kernel(dst_hbm, zeros_hbm, ones_hbm, out_hbm, dst_v, zb, ob, acc):
        c = lax.axis_index("c")
        s = lax.axis_index("s")
        z0 = s * ROWS_PER_TILE

        pltpu.sync_copy(zeros_hbm, zb)
        pltpu.sync_copy(ones_hbm, ob)

        def zero_body(j, carry):
            pltpu.sync_copy(zb, acc.at[pl.ds(z0 + j * CH, CH)])
            return carry
        lax.fori_loop(0, ROWS_PER_TILE // CH, zero_body, 0)
        plsc.subcore_barrier()

        base = (c * NS + s) * CHUNKS_PER_TILE
        pltpu.sync_copy(dst_hbm.at[pl.ds(base, CHUNKS_PER_TILE)], dst_v)

        def edge_body(j, carry):
            pltpu.sync_copy(ob, acc.at[dst_v.at[j]], add=True)
            return carry
        lax.fori_loop(0, CHUNKS_PER_TILE, edge_body, 0)
        plsc.subcore_barrier()

        def out_body(j, carry):
            r0 = z0 + j * CH
            pltpu.sync_copy(acc.at[pl.ds(r0, CH)], zb)
            pltpu.sync_copy(zb, out_hbm.at[c, pl.ds(r0, CH)])
            return carry
        lax.fori_loop(0, ROWS_PER_TILE // CH, out_body, 0)

    return deg_kernel


def _make_agg_kernel(depth):
    mesh = plsc.VectorSubcoreMesh(core_axis_name="c", subcore_axis_name="s")

    @functools.partial(
        pl.kernel,
        out_type=jax.ShapeDtypeStruct((NC, R_ACC, depth), jnp.float32),
        mesh=mesh,
        scratch_types=[
            pltpu.VMEM((CHUNKS_PER_TILE, CH), jnp.int32),
            pltpu.VMEM((CHUNKS_PER_TILE, CH), jnp.int32),
            pltpu.VMEM((CH, depth), jnp.float32),
            pltpu.VMEM_SHARED((R_ACC, depth), jnp.float32),
            pltpu.SemaphoreType.DMA,
        ],
    )
    def agg_kernel(src_hbm, dst_hbm, hp_hbm, zeros_hbm, out_hbm,
                   src_v, dst_v, gb, acc, sem):
        c = lax.axis_index("c")
        s = lax.axis_index("s")
        z0 = s * ROWS_PER_TILE

        pltpu.sync_copy(zeros_hbm, gb)

        def zero_body(j, carry):
            pltpu.sync_copy(gb, acc.at[pl.ds(z0 + j * CH, CH)])
            return carry
        lax.fori_loop(0, ROWS_PER_TILE // CH, zero_body, 0)
        plsc.subcore_barrier()

        base = (c * NS + s) * CHUNKS_PER_TILE
        pltpu.sync_copy(src_hbm.at[pl.ds(base, CHUNKS_PER_TILE)], src_v)
        pltpu.sync_copy(dst_hbm.at[pl.ds(base, CHUNKS_PER_TILE)], dst_v)

        def edge_body(j, carry):
            pltpu.async_copy(hp_hbm.at[src_v.at[j]], gb, sem).wait()
            pltpu.sync_copy(gb, acc.at[dst_v.at[j]], add=True)
            return carry
        lax.fori_loop(0, CHUNKS_PER_TILE, edge_body, 0)
        plsc.subcore_barrier()

        def out_body(j, carry):
            r0 = z0 + j * CH
            pltpu.sync_copy(acc.at[pl.ds(r0, CH)], gb)
            pltpu.sync_copy(gb, out_hbm.at[c, pl.ds(r0, CH)])
            return carry
        lax.fori_loop(0, ROWS_PER_TILE // CH, out_body, 0)

    return agg_kernel


_deg_kernel = _make_deg_kernel()
_agg128 = _make_agg_kernel(H1)
_agg64 = _make_agg_kernel(H3)


def _mm_body(x_ref, w_ref, o_ref):
    o_ref[...] = jnp.dot(x_ref[...], w_ref[...],
                         preferred_element_type=jnp.float32)


def _mm(x, w):
    m, k = x.shape
    n = w.shape[1]
    return pl.pallas_call(
        _mm_body,
        grid=(GRID,),
        in_specs=[
            pl.BlockSpec((BLK, k), lambda i: (i, 0)),
            pl.BlockSpec((k, n), lambda i: (0, 0)),
        ],
        out_specs=pl.BlockSpec((BLK, n), lambda i: (i, 0)),
        out_shape=jax.ShapeDtypeStruct((m, n), jnp.float32),
    )(x, w)


def _scale_body(raw_ref, degp_ref, hp_ref, dis_ref):
    degp = degp_ref[...]
    deg = 1.0 + degp[0, :, 0:1] + degp[1, :, 0:1]
    dis = lax.rsqrt(deg)
    hp_ref[...] = raw_ref[...] * dis
    dis_ref[...] = jnp.broadcast_to(dis, (BLK, 16))


def _scale(raw, degp):
    return pl.pallas_call(
        _scale_body,
        grid=(GRID,),
        in_specs=[
            pl.BlockSpec((BLK, H1), lambda i: (i, 0)),
            pl.BlockSpec((NC, BLK, 16), lambda i: (0, i, 0)),
        ],
        out_specs=[
            pl.BlockSpec((BLK, H1), lambda i: (i, 0)),
            pl.BlockSpec((BLK, 16), lambda i: (i, 0)),
        ],
        out_shape=[
            jax.ShapeDtypeStruct((N, H1), jnp.float32),
            jax.ShapeDtypeStruct((N, 16), jnp.float32),
        ],
    )(raw, degp)


def _stats_body(p_ref, hp_ref, dis_ref, b_ref, t_ref, s_ref, ss_ref):
    p = p_ref[...]
    t = (p[0] + p[1] + hp_ref[...]) * dis_ref[:, 0:1] + b_ref[...]
    t_ref[...] = t

    @pl.when(pl.program_id(0) == 0)
    def _():
        s_ref[...] = jnp.zeros_like(s_ref)
        ss_ref[...] = jnp.zeros_like(ss_ref)

    s_ref[...] += jnp.sum(t, axis=0, keepdims=True)
    ss_ref[...] += jnp.sum(t * t, axis=0, keepdims=True)


def _stats(p, hp, dis, b):
    depth = hp.shape[1]
    return pl.pallas_call(
        _stats_body,
        grid=(GRID,),
        in_specs=[
            pl.BlockSpec((NC, BLK, depth), lambda i: (0, i, 0)),
            pl.BlockSpec((BLK, depth), lambda i: (i, 0)),
            pl.BlockSpec((BLK, 16), lambda i: (i, 0)),
            pl.BlockSpec((1, depth), lambda i: (0, 0)),
        ],
        out_specs=[
            pl.BlockSpec((BLK, depth), lambda i: (i, 0)),
            pl.BlockSpec((1, depth), lambda i: (0, 0)),
            pl.BlockSpec((1, depth), lambda i: (0, 0)),
        ],
        out_shape=[
            jax.ShapeDtypeStruct((N, depth), jnp.float32),
            jax.ShapeDtypeStruct((1, depth), jnp.float32),
            jax.ShapeDtypeStruct((1, depth), jnp.float32),
        ],
    )(p, hp, dis, b)


def _bnmm_body(t_ref, s_ref, ss_ref, g_ref, bt_ref, w_ref, dis_ref, o_ref):
    m = s_ref[...] * (1.0 / N)
    v = ss_ref[...] * (1.0 / N) - m * m
    y = (t_ref[...] - m) * lax.rsqrt(v + 1e-5) * g_ref[...] + bt_ref[...]
    y = jnp.maximum(y, 0.0)
    o_ref[...] = jnp.dot(y, w_ref[...],
                         preferred_element_type=jnp.float32) * dis_ref[:, 0:1]


def _bnmm(t, s, ss, g, bt, w, dis):
    depth = t.shape[1]
    dn = w.shape[1]
    return pl.pallas_call(
        _bnmm_body,
        grid=(GRID,),
        in_specs=[
            pl.BlockSpec((BLK, depth), lambda i: (i, 0)),
            pl.BlockSpec((1, depth), lambda i: (0, 0)),
            pl.BlockSpec((1, depth), lambda i: (0, 0)),
            pl.BlockSpec((1, depth), lambda i: (0, 0)),
            pl.BlockSpec((1, depth), lambda i: (0, 0)),
            pl.BlockSpec((depth, dn), lambda i: (0, 0)),
            pl.BlockSpec((BLK, 16), lambda i: (i, 0)),
        ],
        out_specs=pl.BlockSpec((BLK, dn), lambda i: (i, 0)),
        out_shape=jax.ShapeDtypeStruct((N, dn), jnp.float32),
    )(t, s, ss, g, bt, w, dis)


def _head_body(t_ref, s_ref, ss_ref, g_ref, bt_ref, cw1_ref, cb1_ref,
               cw2t_ref, cb2_ref, o_ref):
    m = s_ref[...] * (1.0 / N)
    v = ss_ref[...] * (1.0 / N) - m * m
    y = (t_ref[...] - m) * lax.rsqrt(v + 1e-5) * g_ref[...] + bt_ref[...]
    y = jnp.maximum(y, 0.0)
    h = jnp.dot(y, cw1_ref[...], preferred_element_type=jnp.float32)
    h = jnp.maximum(h + cb1_ref[...], 0.0)
    o_ref[...] = (jnp.sum(h * cw2t_ref[...], axis=1, keepdims=True)
                  + cb2_ref[...])


def _head(t, s, ss, g, bt, cw1, cb1, cw2t, cb2):
    return pl.pallas_call(
        _head_body,
        grid=(GRID,),
        in_specs=[
            pl.BlockSpec((BLK, H3), lambda i: (i, 0)),
            pl.BlockSpec((1, H3), lambda i: (0, 0)),
            pl.BlockSpec((1, H3), lambda i: (0, 0)),
            pl.BlockSpec((1, H3), lambda i: (0, 0)),
            pl.BlockSpec((1, H3), lambda i: (0, 0)),
            pl.BlockSpec((H3, 32), lambda i: (0, 0)),
            pl.BlockSpec((1, 32), lambda i: (0, 0)),
            pl.BlockSpec((1, 32), lambda i: (0, 0)),
            pl.BlockSpec((1, 1), lambda i: (0, 0)),
        ],
        out_specs=pl.BlockSpec((BLK, 1), lambda i: (i, 0)),
        out_shape=jax.ShapeDtypeStruct((N, 1), jnp.float32),
    )(t, s, ss, g, bt, cw1, cb1, cw2t, cb2)


@jax.jit
def _run(x, edge_index, W1, b1, g1, bt1, W2, b2, g2, bt2, W3, b3, g3, bt3,
         cW1, cb1, cW2, cb2):
    src = edge_index[0]
    dst = edge_index[1]
    pad = EP - E
    src_p = jnp.concatenate([src, jnp.zeros((pad,), jnp.int32)])
    dst_p = jnp.concatenate(
        [dst, N + (jnp.arange(pad, dtype=jnp.int32) % (R_ACC - N))])
    src2d = src_p.reshape(CHUNKS, CH)
    dst2d = dst_p.reshape(CHUNKS, CH)

    zeros16 = jnp.zeros((CH, 16), jnp.float32)
    ones16 = jnp.ones((CH, 16), jnp.float32)
    zeros128 = jnp.zeros((CH, H1), jnp.float32)
    zeros64 = jnp.zeros((CH, H3), jnp.float32)

    b1r = b1.reshape(1, H1)
    g1r = g1.reshape(1, H1)
    bt1r = bt1.reshape(1, H1)
    b2r = b2.reshape(1, H1)
    g2r = g2.reshape(1, H1)
    bt2r = bt2.reshape(1, H1)
    b3r = b3.reshape(1, H3)
    g3r = g3.reshape(1, H3)
    bt3r = bt3.reshape(1, H3)
    cb1r = cb1.reshape(1, 32)
    cw2t = cW2.reshape(1, 32)
    cb2r = cb2.reshape(1, 1)

    degp = _deg_kernel(dst2d, zeros16, ones16)
    raw1 = _mm(x, W1)
    hp1, dis = _scale(raw1, degp)

    p1 = _agg128(src2d, dst2d, hp1, zeros128)
    t1, s1, ss1 = _stats(p1, hp1, dis, b1r)
    hp2 = _bnmm(t1, s1, ss1, g1r, bt1r, W2, dis)

    p2 = _agg128(src2d, dst2d, hp2, zeros128)
    t2, s2, ss2 = _stats(p2, hp2, dis, b2r)
    hp3 = _bnmm(t2, s2, ss2, g2r, bt2r, W3, dis)

    p3 = _agg64(src2d, dst2d, hp3, zeros64)
    t3, s3, ss3 = _stats(p3, hp3, dis, b3r)
    out = _head(t3, s3, ss3, g3r, bt3r, cW1, cb1r, cw2t, cb2r)
    return out[:, 0]


def kernel(x, edge_index, W1, b1, g1, bt1, W2, b2, g2, bt2, W3, b3, g3, bt3,
           cW1, cb1, cW2, cb2):
    return _run(x, edge_index, W1, b1, g1, bt1, W2, b2, g2, bt2,
                W3, b3, g3, bt3, cW1, cb1, cW2, cb2)


# speedup vs baseline: 8.2517x; 8.2517x over previous
import functools

import jax
import jax.numpy as jnp
from jax import lax
from jax.experimental import pallas as pl
from jax.experimental.pallas import tpu as pltpu
from jax.experimental.pallas import tpu_sc as plsc

N = 10000
E = 320000
D_IN = 128
H1 = 128
H3 = 64

NC = 2
NS = 16
CH = 128

R_ACC = 10240
ROWS_PER_TILE = R_ACC // NS
EDGES_PER_SC = ((E // NC + NS * CH * 8 - 1) // (NS * CH * 8)) * (NS * CH * 8)
EP = EDGES_PER_SC * NC
CHUNKS = EP // CH
CHUNKS_PER_TILE = CHUNKS // (NC * NS)

BLK = 1000
GRID = N // BLK


def _make_deg_kernel():
    mesh = plsc.VectorSubcoreMesh(core_axis_name="c", subcore_axis_name="s")

    @functools.partial(
        pl.kernel,
        out_type=jax.ShapeDtypeStruct((NC, R_ACC, 16), jnp.float32),
        mesh=mesh,
        scratch_types=[
            pltpu.VMEM((CHUNKS_PER_TILE, CH), jnp.int32),
            pltpu.VMEM((CH, 16), jnp.float32),
            pltpu.VMEM((CH, 16), jnp.float32),
            pltpu.VMEM_SHARED((R_ACC, 16), jnp.float32),
        ],
        compiler_params=pltpu.CompilerParams(use_tc_tiling_on_sc=False),
    )
    def deg_kernel(dst_hbm, zeros_hbm, ones_hbm, out_hbm, dst_v, zb, ob, acc):
        c = lax.axis_index("c")
        s = lax.axis_index("s")
        z0 = s * ROWS_PER_TILE

        pltpu.sync_copy(zeros_hbm, zb)
        pltpu.sync_copy(ones_hbm, ob)

        def zero_body(j, carry):
            pltpu.sync_copy(zb, acc.at[pl.ds(z0 + j * CH, CH)])
            return carry
        lax.fori_loop(0, ROWS_PER_TILE // CH, zero_body, 0)
        plsc.subcore_barrier()

        base = (c * NS + s) * CHUNKS_PER_TILE
        pltpu.sync_copy(dst_hbm.at[pl.ds(base, CHUNKS_PER_TILE)], dst_v)

        def edge_body(j, carry):
            pltpu.sync_copy(ob, acc.at[dst_v.at[j]], add=True)
            return carry
        lax.fori_loop(0, CHUNKS_PER_TILE, edge_body, 0)
        plsc.subcore_barrier()

        def out_body(j, carry):
            r0 = z0 + j * CH
            pltpu.sync_copy(acc.at[pl.ds(r0, CH)], zb)
            pltpu.sync_copy(zb, out_hbm.at[c, pl.ds(r0, CH)])
            return carry
        lax.fori_loop(0, ROWS_PER_TILE // CH, out_body, 0)

    return deg_kernel


def _make_agg_kernel(depth):
    mesh = plsc.VectorSubcoreMesh(core_axis_name="c", subcore_axis_name="s")

    @functools.partial(
        pl.kernel,
        out_type=jax.ShapeDtypeStruct((NC, R_ACC, depth), jnp.float32),
        mesh=mesh,
        scratch_types=[
            pltpu.VMEM((CHUNKS_PER_TILE, CH), jnp.int32),
            pltpu.VMEM((CHUNKS_PER_TILE, CH), jnp.int32),
            pltpu.VMEM((CH, depth), jnp.float32),
            pltpu.VMEM_SHARED((R_ACC, depth), jnp.float32),
            pltpu.SemaphoreType.DMA,
        ],
        compiler_params=pltpu.CompilerParams(use_tc_tiling_on_sc=False),
    )
    def agg_kernel(src_hbm, dst_hbm, hp_hbm, zeros_hbm, out_hbm,
                   src_v, dst_v, gb, acc, sem):
        c = lax.axis_index("c")
        s = lax.axis_index("s")
        z0 = s * ROWS_PER_TILE

        pltpu.sync_copy(zeros_hbm, gb)

        def zero_body(j, carry):
            pltpu.sync_copy(gb, acc.at[pl.ds(z0 + j * CH, CH)])
            return carry
        lax.fori_loop(0, ROWS_PER_TILE // CH, zero_body, 0)
        plsc.subcore_barrier()

        base = (c * NS + s) * CHUNKS_PER_TILE
        pltpu.sync_copy(src_hbm.at[pl.ds(base, CHUNKS_PER_TILE)], src_v)
        pltpu.sync_copy(dst_hbm.at[pl.ds(base, CHUNKS_PER_TILE)], dst_v)

        def edge_body(j, carry):
            pltpu.async_copy(hp_hbm.at[src_v.at[j]], gb, sem).wait()
            pltpu.sync_copy(gb, acc.at[dst_v.at[j]], add=True)
            return carry
        lax.fori_loop(0, CHUNKS_PER_TILE, edge_body, 0)
        plsc.subcore_barrier()

        def out_body(j, carry):
            r0 = z0 + j * CH
            pltpu.sync_copy(acc.at[pl.ds(r0, CH)], gb)
            pltpu.sync_copy(gb, out_hbm.at[c, pl.ds(r0, CH)])
            return carry
        lax.fori_loop(0, ROWS_PER_TILE // CH, out_body, 0)

    return agg_kernel


_deg_kernel = _make_deg_kernel()
_agg128 = _make_agg_kernel(H1)
_agg64 = _make_agg_kernel(H3)


def _mm_body(x_ref, w_ref, o_ref):
    o_ref[...] = jnp.dot(x_ref[...], w_ref[...],
                         preferred_element_type=jnp.float32)


def _mm(x, w):
    m, k = x.shape
    n = w.shape[1]
    return pl.pallas_call(
        _mm_body,
        grid=(GRID,),
        in_specs=[
            pl.BlockSpec((BLK, k), lambda i: (i, 0)),
            pl.BlockSpec((k, n), lambda i: (0, 0)),
        ],
        out_specs=pl.BlockSpec((BLK, n), lambda i: (i, 0)),
        out_shape=jax.ShapeDtypeStruct((m, n), jnp.float32),
    )(x, w)


def _scale_body(raw_ref, degp_ref, hp_ref, dis_ref):
    degp = degp_ref[...]
    deg = 1.0 + degp[0, :, 0:1] + degp[1, :, 0:1]
    dis = lax.rsqrt(deg)
    hp_ref[...] = raw_ref[...] * dis
    dis_ref[...] = jnp.broadcast_to(dis, (BLK, 16))


def _scale(raw, degp):
    return pl.pallas_call(
        _scale_body,
        grid=(GRID,),
        in_specs=[
            pl.BlockSpec((BLK, H1), lambda i: (i, 0)),
            pl.BlockSpec((NC, BLK, 16), lambda i: (0, i, 0)),
        ],
        out_specs=[
            pl.BlockSpec((BLK, H1), lambda i: (i, 0)),
            pl.BlockSpec((BLK, 16), lambda i: (i, 0)),
        ],
        out_shape=[
            jax.ShapeDtypeStruct((N, H1), jnp.float32),
            jax.ShapeDtypeStruct((N, 16), jnp.float32),
        ],
    )(raw, degp)


def _stats_body(p_ref, hp_ref, dis_ref, b_ref, t_ref, s_ref, ss_ref):
    p = p_ref[...]
    t = (p[0] + p[1] + hp_ref[...]) * dis_ref[:, 0:1] + b_ref[...]
    t_ref[...] = t

    @pl.when(pl.program_id(0) == 0)
    def _():
        s_ref[...] = jnp.zeros_like(s_ref)
        ss_ref[...] = jnp.zeros_like(ss_ref)

    s_ref[...] += jnp.sum(t, axis=0, keepdims=True)
    ss_ref[...] += jnp.sum(t * t, axis=0, keepdims=True)


def _stats(p, hp, dis, b):
    depth = hp.shape[1]
    return pl.pallas_call(
        _stats_body,
        grid=(GRID,),
        in_specs=[
            pl.BlockSpec((NC, BLK, depth), lambda i: (0, i, 0)),
            pl.BlockSpec((BLK, depth), lambda i: (i, 0)),
            pl.BlockSpec((BLK, 16), lambda i: (i, 0)),
            pl.BlockSpec((1, depth), lambda i: (0, 0)),
        ],
        out_specs=[
            pl.BlockSpec((BLK, depth), lambda i: (i, 0)),
            pl.BlockSpec((1, depth), lambda i: (0, 0)),
            pl.BlockSpec((1, depth), lambda i: (0, 0)),
        ],
        out_shape=[
            jax.ShapeDtypeStruct((N, depth), jnp.float32),
            jax.ShapeDtypeStruct((1, depth), jnp.float32),
            jax.ShapeDtypeStruct((1, depth), jnp.float32),
        ],
    )(p, hp, dis, b)


def _bnmm_body(t_ref, s_ref, ss_ref, g_ref, bt_ref, w_ref, dis_ref, o_ref):
    m = s_ref[...] * (1.0 / N)
    v = ss_ref[...] * (1.0 / N) - m * m
    y = (t_ref[...] - m) * lax.rsqrt(v + 1e-5) * g_ref[...] + bt_ref[...]
    y = jnp.maximum(y, 0.0)
    o_ref[...] = jnp.dot(y, w_ref[...],
                         preferred_element_type=jnp.float32) * dis_ref[:, 0:1]


def _bnmm(t, s, ss, g, bt, w, dis):
    depth = t.shape[1]
    dn = w.shape[1]
    return pl.pallas_call(
        _bnmm_body,
        grid=(GRID,),
        in_specs=[
            pl.BlockSpec((BLK, depth), lambda i: (i, 0)),
            pl.BlockSpec((1, depth), lambda i: (0, 0)),
            pl.BlockSpec((1, depth), lambda i: (0, 0)),
            pl.BlockSpec((1, depth), lambda i: (0, 0)),
            pl.BlockSpec((1, depth), lambda i: (0, 0)),
            pl.BlockSpec((depth, dn), lambda i: (0, 0)),
            pl.BlockSpec((BLK, 16), lambda i: (i, 0)),
        ],
        out_specs=pl.BlockSpec((BLK, dn), lambda i: (i, 0)),
        out_shape=jax.ShapeDtypeStruct((N, dn), jnp.float32),
    )(t, s, ss, g, bt, w, dis)


def _head_body(t_ref, s_ref, ss_ref, g_ref, bt_ref, cw1_ref, cb1_ref,
               cw2t_ref, cb2_ref, o_ref):
    m = s_ref[...] * (1.0 / N)
    v = ss_ref[...] * (1.0 / N) - m * m
    y = (t_ref[...] - m) * lax.rsqrt(v + 1e-5) * g_ref[...] + bt_ref[...]
    y = jnp.maximum(y, 0.0)
    h = jnp.dot(y, cw1_ref[...], preferred_element_type=jnp.float32)
    h = jnp.maximum(h + cb1_ref[...], 0.0)
    o_ref[...] = (jnp.sum(h * cw2t_ref[...], axis=1, keepdims=True)
                  + cb2_ref[...])


def _head(t, s, ss, g, bt, cw1, cb1, cw2t, cb2):
    return pl.pallas_call(
        _head_body,
        grid=(GRID,),
        in_specs=[
            pl.BlockSpec((BLK, H3), lambda i: (i, 0)),
            pl.BlockSpec((1, H3), lambda i: (0, 0)),
            pl.BlockSpec((1, H3), lambda i: (0, 0)),
            pl.BlockSpec((1, H3), lambda i: (0, 0)),
            pl.BlockSpec((1, H3), lambda i: (0, 0)),
            pl.BlockSpec((H3, 32), lambda i: (0, 0)),
            pl.BlockSpec((1, 32), lambda i: (0, 0)),
            pl.BlockSpec((1, 32), lambda i: (0, 0)),
            pl.BlockSpec((1, 1), lambda i: (0, 0)),
        ],
        out_specs=pl.BlockSpec((BLK, 1), lambda i: (i, 0)),
        out_shape=jax.ShapeDtypeStruct((N, 1), jnp.float32),
    )(t, s, ss, g, bt, cw1, cb1, cw2t, cb2)


@jax.jit
def _run(x, edge_index, W1, b1, g1, bt1, W2, b2, g2, bt2, W3, b3, g3, bt3,
         cW1, cb1, cW2, cb2):
    src = edge_index[0]
    dst = edge_index[1]
    pad = EP - E
    src_p = jnp.concatenate([src, jnp.zeros((pad,), jnp.int32)])
    dst_p = jnp.concatenate(
        [dst, N + (jnp.arange(pad, dtype=jnp.int32) % (R_ACC - N))])
    src2d = src_p.reshape(CHUNKS, CH)
    dst2d = dst_p.reshape(CHUNKS, CH)

    zeros16 = jnp.zeros((CH, 16), jnp.float32)
    ones16 = jnp.ones((CH, 16), jnp.float32)
    zeros128 = jnp.zeros((CH, H1), jnp.float32)
    zeros64 = jnp.zeros((CH, H3), jnp.float32)

    b1r = b1.reshape(1, H1)
    g1r = g1.reshape(1, H1)
    bt1r = bt1.reshape(1, H1)
    b2r = b2.reshape(1, H1)
    g2r = g2.reshape(1, H1)
    bt2r = bt2.reshape(1, H1)
    b3r = b3.reshape(1, H3)
    g3r = g3.reshape(1, H3)
    bt3r = bt3.reshape(1, H3)
    cb1r = cb1.reshape(1, 32)
    cw2t = cW2.reshape(1, 32)
    cb2r = cb2.reshape(1, 1)

    degp = _deg_kernel(dst2d, zeros16, ones16)
    raw1 = _mm(x, W1)
    hp1, dis = _scale(raw1, degp)

    p1 = _agg128(src2d, dst2d, hp1, zeros128)
    t1, s1, ss1 = _stats(p1, hp1, dis, b1r)
    hp2 = _bnmm(t1, s1, ss1, g1r, bt1r, W2, dis)

    p2 = _agg128(src2d, dst2d, hp2, zeros128)
    t2, s2, ss2 = _stats(p2, hp2, dis, b2r)
    hp3 = _bnmm(t2, s2, ss2, g2r, bt2r, W3, dis)

    p3 = _agg64(src2d, dst2d, hp3, zeros64)
    t3, s3, ss3 = _stats(p3, hp3, dis, b3r)
    out = _head(t3, s3, ss3, g3r, bt3r, cW1, cb1r, cw2t, cb2r)
    return out[:, 0]


def kernel(x, edge_index, W1, b1, g1, bt1, W2, b2, g2, bt2, W3, b3, g3, bt3,
           cW1, cb1, cW2, cb2):
    return _run(x, edge_index, W1, b1, g1, bt1, W2, b2, g2, bt2,
                W3, b3, g3, bt3, cW1, cb1, cW2, cb2)


# speedup vs baseline: 9.2191x; 1.1172x over previous
import functools

import jax
import jax.numpy as jnp
from jax import lax
from jax.experimental import pallas as pl
from jax.experimental.pallas import tpu as pltpu
from jax.experimental.pallas import tpu_sc as plsc

N = 10000
E = 320000
D_IN = 128
H1 = 128
H3 = 64

NC = 2
NS = 16
CH = 128

R_ACC = 10240
ROWS_PER_TILE = R_ACC // NS
EDGES_PER_SC = ((E // NC + NS * CH * 8 - 1) // (NS * CH * 8)) * (NS * CH * 8)
EP = EDGES_PER_SC * NC
CHUNKS = EP // CH
CHUNKS_PER_TILE = CHUNKS // (NC * NS)

BLK = 1000
GRID = N // BLK


def _make_deg_kernel():
    mesh = plsc.VectorSubcoreMesh(core_axis_name="c", subcore_axis_name="s")

    @functools.partial(
        pl.kernel,
        out_type=jax.ShapeDtypeStruct((NC, R_ACC, 16), jnp.float32),
        mesh=mesh,
        scratch_types=[
            pltpu.VMEM((CHUNKS_PER_TILE, CH), jnp.int32),
            pltpu.VMEM((CH, 16), jnp.float32),
            pltpu.VMEM((CH, 16), jnp.float32),
            pltpu.VMEM_SHARED((R_ACC, 16), jnp.float32),
        ],
        compiler_params=pltpu.CompilerParams(use_tc_tiling_on_sc=False),
    )
    def deg_kernel(dst_hbm, zeros_hbm, ones_hbm, out_hbm, dst_v, zb, ob, acc):
        c = lax.axis_index("c")
        s = lax.axis_index("s")
        z0 = s * ROWS_PER_TILE

        pltpu.sync_copy(zeros_hbm, zb)
        pltpu.sync_copy(ones_hbm, ob)

        def zero_body(j, carry):
            pltpu.sync_copy(zb, acc.at[pl.ds(z0 + j * CH, CH)])
            return carry
        lax.fori_loop(0, ROWS_PER_TILE // CH, zero_body, 0)
        plsc.subcore_barrier()

        base = (c * NS + s) * CHUNKS_PER_TILE
        pltpu.sync_copy(dst_hbm.at[pl.ds(base, CHUNKS_PER_TILE)], dst_v)

        def edge_body(j, carry):
            pltpu.sync_copy(ob, acc.at[dst_v.at[j]], add=True)
            return carry
        lax.fori_loop(0, CHUNKS_PER_TILE, edge_body, 0)
        plsc.subcore_barrier()

        def out_body(j, carry):
            r0 = z0 + j * CH
            pltpu.sync_copy(acc.at[pl.ds(r0, CH)], zb)
            pltpu.sync_copy(zb, out_hbm.at[c, pl.ds(r0, CH)])
            return carry
        lax.fori_loop(0, ROWS_PER_TILE // CH, out_body, 0)

    return deg_kernel


NBUF = 2
PASS = 16
NPASS = CHUNKS_PER_TILE // PASS
PGRP = PASS // NBUF


def _make_agg_kernel(depth):
    mesh = plsc.VectorSubcoreMesh(core_axis_name="c", subcore_axis_name="s")

    @functools.partial(
        pl.kernel,
        out_type=jax.ShapeDtypeStruct((NC, R_ACC, depth), jnp.float32),
        mesh=mesh,
        scratch_types=(
            [pltpu.VMEM((PASS, CH), jnp.int32)] * 4
            + [pltpu.VMEM((CH, depth), jnp.float32)] * NBUF
            + [pltpu.VMEM_SHARED((R_ACC, depth), jnp.float32)]
            + [pltpu.SemaphoreType.DMA] * (NBUF + 2)
        ),
        compiler_params=pltpu.CompilerParams(use_tc_tiling_on_sc=False),
    )
    def agg_kernel(src_hbm, dst_hbm, hp_hbm, zeros_hbm, out_hbm, *rest):
        idx = rest[:4]
        gbs = rest[4:4 + NBUF]
        acc = rest[4 + NBUF]
        gsems = rest[5 + NBUF:5 + 2 * NBUF]
        isems = rest[5 + 2 * NBUF:]
        c = lax.axis_index("c")
        s = lax.axis_index("s")
        z0 = s * ROWS_PER_TILE
        base = (c * NS + s) * CHUNKS_PER_TILE

        pltpu.sync_copy(zeros_hbm, gbs[0])

        def zero_body(j, carry):
            pltpu.sync_copy(gbs[0], acc.at[pl.ds(z0 + j * CH, CH)])
            return carry
        lax.fori_loop(0, ROWS_PER_TILE // CH, zero_body, 0)

        pltpu.sync_copy(src_hbm.at[pl.ds(base, PASS)], idx[0])
        pltpu.sync_copy(dst_hbm.at[pl.ds(base, PASS)], idx[1])
        for b in range(NBUF):
            pltpu.async_copy(hp_hbm.at[idx[0].at[b]], gbs[b], gsems[b])
        plsc.subcore_barrier()

        for p in range(NPASS):
            sv, dv = idx[2 * (p % 2)], idx[2 * (p % 2) + 1]
            nsv, ndv = idx[2 * ((p + 1) % 2)], idx[2 * ((p + 1) % 2) + 1]
            if p + 1 < NPASS:
                nb = base + (p + 1) * PASS
                pltpu.async_copy(src_hbm.at[pl.ds(nb, PASS)], nsv, isems[0])
                pltpu.async_copy(dst_hbm.at[pl.ds(nb, PASS)], ndv, isems[1])
            if p > 0:
                for b in range(NBUF):
                    pltpu.async_copy(hp_hbm.at[sv.at[b]], gbs[b], gsems[b])

            def grp_body(grp, carry):
                t0 = grp * NBUF
                for b in range(NBUF):
                    pltpu.make_async_copy(
                        hp_hbm.at[sv.at[t0 + b]], gbs[b], gsems[b]).wait()
                    pltpu.sync_copy(gbs[b], acc.at[dv.at[t0 + b]], add=True)

                    @pl.when(grp + 1 < PGRP)
                    def _():
                        pltpu.async_copy(
                            hp_hbm.at[sv.at[t0 + NBUF + b]], gbs[b], gsems[b])
                return carry
            lax.fori_loop(0, PGRP, grp_body, 0)

            if p + 1 < NPASS:
                nb = base + (p + 1) * PASS
                pltpu.make_async_copy(
                    src_hbm.at[pl.ds(nb, PASS)], nsv, isems[0]).wait()
                pltpu.make_async_copy(
                    dst_hbm.at[pl.ds(nb, PASS)], ndv, isems[1]).wait()
        plsc.subcore_barrier()

        def out_body(j, carry):
            r0 = z0 + j * CH
            pltpu.sync_copy(acc.at[pl.ds(r0, CH)], gbs[0])
            pltpu.sync_copy(gbs[0], out_hbm.at[c, pl.ds(r0, CH)])
            return carry
        lax.fori_loop(0, ROWS_PER_TILE // CH, out_body, 0)

    return agg_kernel


_deg_kernel = _make_deg_kernel()
_agg128 = _make_agg_kernel(H1)
_agg64 = _make_agg_kernel(H3)


def _mm_body(x_ref, w_ref, o_ref):
    o_ref[...] = jnp.dot(x_ref[...], w_ref[...],
                         preferred_element_type=jnp.float32)


def _mm(x, w):
    m, k = x.shape
    n = w.shape[1]
    return pl.pallas_call(
        _mm_body,
        grid=(GRID,),
        in_specs=[
            pl.BlockSpec((BLK, k), lambda i: (i, 0)),
            pl.BlockSpec((k, n), lambda i: (0, 0)),
        ],
        out_specs=pl.BlockSpec((BLK, n), lambda i: (i, 0)),
        out_shape=jax.ShapeDtypeStruct((m, n), jnp.float32),
    )(x, w)


def _scale_body(raw_ref, degp_ref, hp_ref, dis_ref):
    degp = degp_ref[...]
    deg = 1.0 + degp[0, :, 0:1] + degp[1, :, 0:1]
    dis = lax.rsqrt(deg)
    hp_ref[...] = raw_ref[...] * dis
    dis_ref[...] = jnp.broadcast_to(dis, (BLK, 16))


def _scale(raw, degp):
    return pl.pallas_call(
        _scale_body,
        grid=(GRID,),
        in_specs=[
            pl.BlockSpec((BLK, H1), lambda i: (i, 0)),
            pl.BlockSpec((NC, BLK, 16), lambda i: (0, i, 0)),
        ],
        out_specs=[
            pl.BlockSpec((BLK, H1), lambda i: (i, 0)),
            pl.BlockSpec((BLK, 16), lambda i: (i, 0)),
        ],
        out_shape=[
            jax.ShapeDtypeStruct((N, H1), jnp.float32),
            jax.ShapeDtypeStruct((N, 16), jnp.float32),
        ],
    )(raw, degp)


def _stats_body(p_ref, hp_ref, dis_ref, b_ref, t_ref, s_ref, ss_ref):
    p = p_ref[...]
    t = (p[0] + p[1] + hp_ref[...]) * dis_ref[:, 0:1] + b_ref[...]
    t_ref[...] = t

    @pl.when(pl.program_id(0) == 0)
    def _():
        s_ref[...] = jnp.zeros_like(s_ref)
        ss_ref[...] = jnp.zeros_like(ss_ref)

    s_ref[...] += jnp.sum(t, axis=0, keepdims=True)
    ss_ref[...] += jnp.sum(t * t, axis=0, keepdims=True)


def _stats(p, hp, dis, b):
    depth = hp.shape[1]
    return pl.pallas_call(
        _stats_body,
        grid=(GRID,),
        in_specs=[
            pl.BlockSpec((NC, BLK, depth), lambda i: (0, i, 0)),
            pl.BlockSpec((BLK, depth), lambda i: (i, 0)),
            pl.BlockSpec((BLK, 16), lambda i: (i, 0)),
            pl.BlockSpec((1, depth), lambda i: (0, 0)),
        ],
        out_specs=[
            pl.BlockSpec((BLK, depth), lambda i: (i, 0)),
            pl.BlockSpec((1, depth), lambda i: (0, 0)),
            pl.BlockSpec((1, depth), lambda i: (0, 0)),
        ],
        out_shape=[
            jax.ShapeDtypeStruct((N, depth), jnp.float32),
            jax.ShapeDtypeStruct((1, depth), jnp.float32),
            jax.ShapeDtypeStruct((1, depth), jnp.float32),
        ],
    )(p, hp, dis, b)


def _bnmm_body(t_ref, s_ref, ss_ref, g_ref, bt_ref, w_ref, dis_ref, o_ref):
    m = s_ref[...] * (1.0 / N)
    v = ss_ref[...] * (1.0 / N) - m * m
    y = (t_ref[...] - m) * lax.rsqrt(v + 1e-5) * g_ref[...] + bt_ref[...]
    y = jnp.maximum(y, 0.0)
    o_ref[...] = jnp.dot(y, w_ref[...],
                         preferred_element_type=jnp.float32) * dis_ref[:, 0:1]


def _bnmm(t, s, ss, g, bt, w, dis):
    depth = t.shape[1]
    dn = w.shape[1]
    return pl.pallas_call(
        _bnmm_body,
        grid=(GRID,),
        in_specs=[
            pl.BlockSpec((BLK, depth), lambda i: (i, 0)),
            pl.BlockSpec((1, depth), lambda i: (0, 0)),
            pl.BlockSpec((1, depth), lambda i: (0, 0)),
            pl.BlockSpec((1, depth), lambda i: (0, 0)),
            pl.BlockSpec((1, depth), lambda i: (0, 0)),
            pl.BlockSpec((depth, dn), lambda i: (0, 0)),
            pl.BlockSpec((BLK, 16), lambda i: (i, 0)),
        ],
        out_specs=pl.BlockSpec((BLK, dn), lambda i: (i, 0)),
        out_shape=jax.ShapeDtypeStruct((N, dn), jnp.float32),
    )(t, s, ss, g, bt, w, dis)


def _head_body(t_ref, s_ref, ss_ref, g_ref, bt_ref, cw1_ref, cb1_ref,
               cw2t_ref, cb2_ref, o_ref):
    m = s_ref[...] * (1.0 / N)
    v = ss_ref[...] * (1.0 / N) - m * m
    y = (t_ref[...] - m) * lax.rsqrt(v + 1e-5) * g_ref[...] + bt_ref[...]
    y = jnp.maximum(y, 0.0)
    h = jnp.dot(y, cw1_ref[...], preferred_element_type=jnp.float32)
    h = jnp.maximum(h + cb1_ref[...], 0.0)
    o_ref[...] = (jnp.sum(h * cw2t_ref[...], axis=1, keepdims=True)
                  + cb2_ref[...])


def _head(t, s, ss, g, bt, cw1, cb1, cw2t, cb2):
    return pl.pallas_call(
        _head_body,
        grid=(GRID,),
        in_specs=[
            pl.BlockSpec((BLK, H3), lambda i: (i, 0)),
            pl.BlockSpec((1, H3), lambda i: (0, 0)),
            pl.BlockSpec((1, H3), lambda i: (0, 0)),
            pl.BlockSpec((1, H3), lambda i: (0, 0)),
            pl.BlockSpec((1, H3), lambda i: (0, 0)),
            pl.BlockSpec((H3, 32), lambda i: (0, 0)),
            pl.BlockSpec((1, 32), lambda i: (0, 0)),
            pl.BlockSpec((1, 32), lambda i: (0, 0)),
            pl.BlockSpec((1, 1), lambda i: (0, 0)),
        ],
        out_specs=pl.BlockSpec((BLK, 1), lambda i: (i, 0)),
        out_shape=jax.ShapeDtypeStruct((N, 1), jnp.float32),
    )(t, s, ss, g, bt, cw1, cb1, cw2t, cb2)


@jax.jit
def _run(x, edge_index, W1, b1, g1, bt1, W2, b2, g2, bt2, W3, b3, g3, bt3,
         cW1, cb1, cW2, cb2):
    src = edge_index[0]
    dst = edge_index[1]
    pad = EP - E
    src_p = jnp.concatenate([src, jnp.zeros((pad,), jnp.int32)])
    dst_p = jnp.concatenate(
        [dst, N + (jnp.arange(pad, dtype=jnp.int32) % (R_ACC - N))])
    src2d = src_p.reshape(CHUNKS, CH)
    dst2d = dst_p.reshape(CHUNKS, CH)

    zeros16 = jnp.zeros((CH, 16), jnp.float32)
    ones16 = jnp.ones((CH, 16), jnp.float32)
    zeros128 = jnp.zeros((CH, H1), jnp.float32)
    zeros64 = jnp.zeros((CH, H3), jnp.float32)

    b1r = b1.reshape(1, H1)
    g1r = g1.reshape(1, H1)
    bt1r = bt1.reshape(1, H1)
    b2r = b2.reshape(1, H1)
    g2r = g2.reshape(1, H1)
    bt2r = bt2.reshape(1, H1)
    b3r = b3.reshape(1, H3)
    g3r = g3.reshape(1, H3)
    bt3r = bt3.reshape(1, H3)
    cb1r = cb1.reshape(1, 32)
    cw2t = cW2.reshape(1, 32)
    cb2r = cb2.reshape(1, 1)

    degp = _deg_kernel(dst2d, zeros16, ones16)
    raw1 = _mm(x, W1)
    hp1, dis = _scale(raw1, degp)

    p1 = _agg128(src2d, dst2d, hp1, zeros128)
    t1, s1, ss1 = _stats(p1, hp1, dis, b1r)
    hp2 = _bnmm(t1, s1, ss1, g1r, bt1r, W2, dis)

    p2 = _agg128(src2d, dst2d, hp2, zeros128)
    t2, s2, ss2 = _stats(p2, hp2, dis, b2r)
    hp3 = _bnmm(t2, s2, ss2, g2r, bt2r, W3, dis)

    p3 = _agg64(src2d, dst2d, hp3, zeros64)
    t3, s3, ss3 = _stats(p3, hp3, dis, b3r)
    out = _head(t3, s3, ss3, g3r, bt3r, cW1, cb1r, cw2t, cb2r)
    return out[:, 0]


def kernel(x, edge_index, W1, b1, g1, bt1, W2, b2, g2, bt2, W3, b3, g3, bt3,
           cW1, cb1, cW2, cb2):
    return _run(x, edge_index, W1, b1, g1, bt1, W2, b2, g2, bt2,
                W3, b3, g3, bt3, cW1, cb1, cW2, cb2)


# speedup vs baseline: 18.5311x; 2.0101x over previous
import functools

import jax
import jax.numpy as jnp
from jax import lax
from jax.experimental import pallas as pl
from jax.experimental.pallas import tpu as pltpu
from jax.experimental.pallas import tpu_sc as plsc

N = 10000
E = 320000
H1 = 128
H3 = 64

NC = 2
NS = 16
CH = 128

R_ACC = 10240
ROWS_PER_TILE = R_ACC // NS
EP = ((E + NS * CH * 8 - 1) // (NS * CH * 8)) * (NS * CH * 8)
CHUNKS = EP // CH
CPT = CHUNKS // NS

NBUF = 4
PASS = 8
NPASS = CPT // PASS
PGRP = PASS // NBUF

BLK = 1000
GRID = N // BLK
CIN_TILES = 10


def _make_deg_kernel():
    mesh = plsc.VectorSubcoreMesh(core_axis_name="c", subcore_axis_name="s")

    @functools.partial(
        pl.kernel,
        out_type=jax.ShapeDtypeStruct((NC, R_ACC, 16), jnp.float32),
        mesh=mesh,
        scratch_types=[
            pltpu.VMEM((CPT // NC, CH), jnp.int32),
            pltpu.VMEM((CH, 16), jnp.float32),
            pltpu.VMEM((CH, 16), jnp.float32),
            pltpu.VMEM_SHARED((R_ACC, 16), jnp.float32),
        ],
        compiler_params=pltpu.CompilerParams(use_tc_tiling_on_sc=False),
    )
    def deg_kernel(dst_hbm, zeros_hbm, ones_hbm, out_hbm, dst_v, zb, ob, acc):
        c = lax.axis_index("c")
        s = lax.axis_index("s")
        z0 = s * ROWS_PER_TILE

        pltpu.sync_copy(zeros_hbm, zb)
        pltpu.sync_copy(ones_hbm, ob)

        def zero_body(j, carry):
            pltpu.sync_copy(zb, acc.at[pl.ds(z0 + j * CH, CH)])
            return carry
        lax.fori_loop(0, ROWS_PER_TILE // CH, zero_body, 0)
        plsc.subcore_barrier()

        base = (c * NS + s) * (CPT // NC)
        pltpu.sync_copy(dst_hbm.at[pl.ds(base, CPT // NC)], dst_v)

        def edge_body(j, carry):
            pltpu.sync_copy(ob, acc.at[dst_v.at[j]], add=True)
            return carry
        lax.fori_loop(0, CPT // NC, edge_body, 0)
        plsc.subcore_barrier()

        def out_body(j, carry):
            r0 = z0 + j * CH
            pltpu.sync_copy(acc.at[pl.ds(r0, CH)], zb)
            pltpu.sync_copy(zb, out_hbm.at[c, pl.ds(r0, CH)])
            return carry
        lax.fori_loop(0, ROWS_PER_TILE // CH, out_body, 0)

    return deg_kernel


def _make_agg_kernel(dh):
    mesh = plsc.VectorSubcoreMesh(core_axis_name="c", subcore_axis_name="s")

    @functools.partial(
        pl.kernel,
        out_type=jax.ShapeDtypeStruct((NC, R_ACC, dh), jnp.float32),
        mesh=mesh,
        scratch_types=(
            [pltpu.VMEM((PASS, CH), jnp.int32)] * 4
            + [pltpu.VMEM((CH, dh), jnp.float32)] * NBUF
            + [pltpu.VMEM_SHARED((N, dh), jnp.float32)]
            + [pltpu.VMEM_SHARED((R_ACC, dh), jnp.float32)]
            + [pltpu.SemaphoreType.DMA] * (NBUF + 2)
        ),
        compiler_params=pltpu.CompilerParams(use_tc_tiling_on_sc=False),
    )
    def agg_kernel(src_hbm, dst_hbm, hp_hbm, zeros_hbm, out_hbm, *rest):
        idx = rest[:4]
        gbs = rest[4:4 + NBUF]
        hp_sp = rest[4 + NBUF]
        acc = rest[5 + NBUF]
        gsems = rest[6 + NBUF:6 + 2 * NBUF]
        isems = rest[6 + 2 * NBUF:]
        c = lax.axis_index("c")
        s = lax.axis_index("s")
        z0 = s * ROWS_PER_TILE
        base = s * CPT

        @pl.when(s < CIN_TILES)
        def _():
            r = s * (N // CIN_TILES)
            pltpu.sync_copy(hp_hbm.at[c, pl.ds(r, N // CIN_TILES)],
                            hp_sp.at[pl.ds(r, N // CIN_TILES)])

        pltpu.sync_copy(zeros_hbm, gbs[0])

        def zero_body(j, carry):
            pltpu.sync_copy(gbs[0], acc.at[pl.ds(z0 + j * CH, CH)])
            return carry
        lax.fori_loop(0, ROWS_PER_TILE // CH, zero_body, 0)

        pltpu.sync_copy(src_hbm.at[pl.ds(base, PASS)], idx[0])
        pltpu.sync_copy(dst_hbm.at[pl.ds(base, PASS)], idx[1])
        plsc.subcore_barrier()
        for b in range(NBUF):
            pltpu.async_copy(hp_sp.at[idx[0].at[b]], gbs[b], gsems[b])

        for p in range(NPASS):
            sv, dv = idx[2 * (p % 2)], idx[2 * (p % 2) + 1]
            nsv, ndv = idx[2 * ((p + 1) % 2)], idx[2 * ((p + 1) % 2) + 1]
            if p + 1 < NPASS:
                nb = base + (p + 1) * PASS
                pltpu.async_copy(src_hbm.at[pl.ds(nb, PASS)], nsv, isems[0])
                pltpu.async_copy(dst_hbm.at[pl.ds(nb, PASS)], ndv, isems[1])
            if p > 0:
                for b in range(NBUF):
                    pltpu.async_copy(hp_sp.at[sv.at[b]], gbs[b], gsems[b])

            def grp_body(grp, carry):
                t0 = grp * NBUF
                for b in range(NBUF):
                    pltpu.make_async_copy(
                        hp_sp.at[sv.at[t0 + b]], gbs[b], gsems[b]).wait()
                    pltpu.sync_copy(gbs[b], acc.at[dv.at[t0 + b]], add=True)

                    @pl.when(grp + 1 < PGRP)
                    def _():
                        pltpu.async_copy(
                            hp_sp.at[sv.at[t0 + NBUF + b]], gbs[b], gsems[b])
                return carry
            lax.fori_loop(0, PGRP, grp_body, 0)

            if p + 1 < NPASS:
                nb = base + (p + 1) * PASS
                pltpu.make_async_copy(
                    src_hbm.at[pl.ds(nb, PASS)], nsv, isems[0]).wait()
                pltpu.make_async_copy(
                    dst_hbm.at[pl.ds(nb, PASS)], ndv, isems[1]).wait()
        plsc.subcore_barrier()

        def out_body(j, carry):
            r0 = z0 + j * CH
            pltpu.sync_copy(acc.at[pl.ds(r0, CH)], gbs[0])
            pltpu.sync_copy(gbs[0], out_hbm.at[c, pl.ds(r0, CH)])
            return carry
        lax.fori_loop(0, ROWS_PER_TILE // CH, out_body, 0)

    return agg_kernel


_deg_kernel = _make_deg_kernel()
_agg64 = _make_agg_kernel(H1 // 2)
_agg32 = _make_agg_kernel(H3 // 2)


def _mm_body(x_ref, w_ref, o_ref):
    o_ref[...] = jnp.dot(x_ref[...], w_ref[...],
                         preferred_element_type=jnp.float32)


def _mm(x, w):
    m, k = x.shape
    n = w.shape[1]
    return pl.pallas_call(
        _mm_body,
        grid=(GRID,),
        in_specs=[
            pl.BlockSpec((BLK, k), lambda i: (i, 0)),
            pl.BlockSpec((k, n), lambda i: (0, 0)),
        ],
        out_specs=pl.BlockSpec((BLK, n), lambda i: (i, 0)),
        out_shape=jax.ShapeDtypeStruct((m, n), jnp.float32),
    )(x, w)


def _scale_body(raw_ref, degp_ref, hp_ref, dis_ref):
    degp = degp_ref[...]
    deg = 1.0 + degp[0, :, 0:1] + degp[1, :, 0:1]
    dis = lax.rsqrt(deg)
    hp = raw_ref[...] * dis
    hp_ref[0] = hp[:, :H1 // 2]
    hp_ref[1] = hp[:, H1 // 2:]
    dis_ref[...] = jnp.broadcast_to(dis, (BLK, 16))


def _scale(raw, degp):
    return pl.pallas_call(
        _scale_body,
        grid=(GRID,),
        in_specs=[
            pl.BlockSpec((BLK, H1), lambda i: (i, 0)),
            pl.BlockSpec((NC, BLK, 16), lambda i: (0, i, 0)),
        ],
        out_specs=[
            pl.BlockSpec((NC, BLK, H1 // 2), lambda i: (0, i, 0)),
            pl.BlockSpec((BLK, 16), lambda i: (i, 0)),
        ],
        out_shape=[
            jax.ShapeDtypeStruct((NC, N, H1 // 2), jnp.float32),
            jax.ShapeDtypeStruct((N, 16), jnp.float32),
        ],
    )(raw, degp)


def _stats_body(p_ref, hp_ref, dis_ref, b_ref, t_ref, s_ref, ss_ref):
    p = p_ref[...]
    hp = hp_ref[...]
    agg = jnp.concatenate([p[0] + hp[0], p[1] + hp[1]], axis=1)
    t = agg * dis_ref[:, 0:1] + b_ref[...]
    t_ref[...] = t

    @pl.when(pl.program_id(0) == 0)
    def _():
        s_ref[...] = jnp.zeros_like(s_ref)
        ss_ref[...] = jnp.zeros_like(ss_ref)

    s_ref[...] += jnp.sum(t, axis=0, keepdims=True)
    ss_ref[...] += jnp.sum(t * t, axis=0, keepdims=True)


def _stats(p, hp, dis, b):
    dh = hp.shape[2]
    depth = 2 * dh
    return pl.pallas_call(
        _stats_body,
        grid=(GRID,),
        in_specs=[
            pl.BlockSpec((NC, BLK, dh), lambda i: (0, i, 0)),
            pl.BlockSpec((NC, BLK, dh), lambda i: (0, i, 0)),
            pl.BlockSpec((BLK, 16), lambda i: (i, 0)),
            pl.BlockSpec((1, depth), lambda i: (0, 0)),
        ],
        out_specs=[
            pl.BlockSpec((BLK, depth), lambda i: (i, 0)),
            pl.BlockSpec((1, depth), lambda i: (0, 0)),
            pl.BlockSpec((1, depth), lambda i: (0, 0)),
        ],
        out_shape=[
            jax.ShapeDtypeStruct((N, depth), jnp.float32),
            jax.ShapeDtypeStruct((1, depth), jnp.float32),
            jax.ShapeDtypeStruct((1, depth), jnp.float32),
        ],
    )(p, hp, dis, b)


def _bnmm_body(t_ref, s_ref, ss_ref, g_ref, bt_ref, w_ref, dis_ref, o_ref):
    m = s_ref[...] * (1.0 / N)
    v = ss_ref[...] * (1.0 / N) - m * m
    y = (t_ref[...] - m) * lax.rsqrt(v + 1e-5) * g_ref[...] + bt_ref[...]
    y = jnp.maximum(y, 0.0)
    o = jnp.dot(y, w_ref[...],
                preferred_element_type=jnp.float32) * dis_ref[:, 0:1]
    dn = o.shape[1]
    o_ref[0] = o[:, :dn // 2]
    o_ref[1] = o[:, dn // 2:]


def _bnmm(t, s, ss, g, bt, w, dis):
    depth = t.shape[1]
    dn = w.shape[1]
    return pl.pallas_call(
        _bnmm_body,
        grid=(GRID,),
        in_specs=[
            pl.BlockSpec((BLK, depth), lambda i: (i, 0)),
            pl.BlockSpec((1, depth), lambda i: (0, 0)),
            pl.BlockSpec((1, depth), lambda i: (0, 0)),
            pl.BlockSpec((1, depth), lambda i: (0, 0)),
            pl.BlockSpec((1, depth), lambda i: (0, 0)),
            pl.BlockSpec((depth, dn), lambda i: (0, 0)),
            pl.BlockSpec((BLK, 16), lambda i: (i, 0)),
        ],
        out_specs=pl.BlockSpec((NC, BLK, dn // 2), lambda i: (0, i, 0)),
        out_shape=jax.ShapeDtypeStruct((NC, N, dn // 2), jnp.float32),
    )(t, s, ss, g, bt, w, dis)


def _head_body(t_ref, s_ref, ss_ref, g_ref, bt_ref, cw1_ref, cb1_ref,
               cw2t_ref, cb2_ref, o_ref):
    m = s_ref[...] * (1.0 / N)
    v = ss_ref[...] * (1.0 / N) - m * m
    y = (t_ref[...] - m) * lax.rsqrt(v + 1e-5) * g_ref[...] + bt_ref[...]
    y = jnp.maximum(y, 0.0)
    h = jnp.dot(y, cw1_ref[...], preferred_element_type=jnp.float32)
    h = jnp.maximum(h + cb1_ref[...], 0.0)
    o_ref[...] = (jnp.sum(h * cw2t_ref[...], axis=1, keepdims=True)
                  + cb2_ref[...])


def _head(t, s, ss, g, bt, cw1, cb1, cw2t, cb2):
    return pl.pallas_call(
        _head_body,
        grid=(GRID,),
        in_specs=[
            pl.BlockSpec((BLK, H3), lambda i: (i, 0)),
            pl.BlockSpec((1, H3), lambda i: (0, 0)),
            pl.BlockSpec((1, H3), lambda i: (0, 0)),
            pl.BlockSpec((1, H3), lambda i: (0, 0)),
            pl.BlockSpec((1, H3), lambda i: (0, 0)),
            pl.BlockSpec((H3, 32), lambda i: (0, 0)),
            pl.BlockSpec((1, 32), lambda i: (0, 0)),
            pl.BlockSpec((1, 32), lambda i: (0, 0)),
            pl.BlockSpec((1, 1), lambda i: (0, 0)),
        ],
        out_specs=pl.BlockSpec((BLK, 1), lambda i: (i, 0)),
        out_shape=jax.ShapeDtypeStruct((N, 1), jnp.float32),
    )(t, s, ss, g, bt, cw1, cb1, cw2t, cb2)


@jax.jit
def _run(x, edge_index, W1, b1, g1, bt1, W2, b2, g2, bt2, W3, b3, g3, bt3,
         cW1, cb1, cW2, cb2):
    src = edge_index[0]
    dst = edge_index[1]
    pad = EP - E
    src_p = jnp.concatenate([src, jnp.zeros((pad,), jnp.int32)])
    dst_p = jnp.concatenate(
        [dst, N + (jnp.arange(pad, dtype=jnp.int32) % (R_ACC - N))])
    src2d = src_p.reshape(CHUNKS, CH)
    dst2d = dst_p.reshape(CHUNKS, CH)

    zeros16 = jnp.zeros((CH, 16), jnp.float32)
    ones16 = jnp.ones((CH, 16), jnp.float32)
    zeros64 = jnp.zeros((CH, H1 // 2), jnp.float32)
    zeros32 = jnp.zeros((CH, H3 // 2), jnp.float32)

    b1r = b1.reshape(1, H1)
    g1r = g1.reshape(1, H1)
    bt1r = bt1.reshape(1, H1)
    b2r = b2.reshape(1, H1)
    g2r = g2.reshape(1, H1)
    bt2r = bt2.reshape(1, H1)
    b3r = b3.reshape(1, H3)
    g3r = g3.reshape(1, H3)
    bt3r = bt3.reshape(1, H3)
    cb1r = cb1.reshape(1, 32)
    cw2t = cW2.reshape(1, 32)
    cb2r = cb2.reshape(1, 1)

    degp = _deg_kernel(dst2d, zeros16, ones16)
    raw1 = _mm(x, W1)
    hp1, dis = _scale(raw1, degp)

    p1 = _agg64(src2d, dst2d, hp1, zeros64)
    t1, s1, ss1 = _stats(p1, hp1, dis, b1r)
    hp2 = _bnmm(t1, s1, ss1, g1r, bt1r, W2, dis)

    p2 = _agg64(src2d, dst2d, hp2, zeros64)
    t2, s2, ss2 = _stats(p2, hp2, dis, b2r)
    hp3 = _bnmm(t2, s2, ss2, g2r, bt2r, W3, dis)

    p3 = _agg32(src2d, dst2d, hp3, zeros32)
    t3, s3, ss3 = _stats(p3, hp3, dis, b3r)
    out = _head(t3, s3, ss3, g3r, bt3r, cW1, cb1r, cw2t, cb2r)
    return out[:, 0]


def kernel(x, edge_index, W1, b1, g1, bt1, W2, b2, g2, bt2, W3, b3, g3, bt3,
           cW1, cb1, cW2, cb2):
    return _run(x, edge_index, W1, b1, g1, bt1, W2, b2, g2, bt2,
                W3, b3, g3, bt3, cW1, cb1, cW2, cb2)


# speedup vs baseline: 19.1711x; 1.0345x over previous
import functools

import jax
import jax.numpy as jnp
from jax import lax
from jax.experimental import pallas as pl
from jax.experimental.pallas import tpu as pltpu
from jax.experimental.pallas import tpu_sc as plsc

N = 10000
E = 320000
H1 = 128
H3 = 64

NC = 2
NS = 16
CH = 128

R_ACC = 10240
ROWS_PER_TILE = R_ACC // NS
EP = ((E + NS * CH * 8 - 1) // (NS * CH * 8)) * (NS * CH * 8)
CHUNKS = EP // CH
CPT = CHUNKS // NS

NBUF = 4
PASS = 16
NPASS = CPT // PASS
PGRP = PASS // NBUF

BLK = 1000
GRID = N // BLK
CIN_TILES = 10


def _make_deg_kernel():
    mesh = plsc.VectorSubcoreMesh(core_axis_name="c", subcore_axis_name="s")

    @functools.partial(
        pl.kernel,
        out_type=jax.ShapeDtypeStruct((NC, R_ACC, 16), jnp.float32),
        mesh=mesh,
        scratch_types=[
            pltpu.VMEM((CPT // NC, CH), jnp.int32),
            pltpu.VMEM((CH, 16), jnp.float32),
            pltpu.VMEM((CH, 16), jnp.float32),
            pltpu.VMEM_SHARED((R_ACC, 16), jnp.float32),
        ],
        compiler_params=pltpu.CompilerParams(use_tc_tiling_on_sc=False),
    )
    def deg_kernel(dst_hbm, zeros_hbm, ones_hbm, out_hbm, dst_v, zb, ob, acc):
        c = lax.axis_index("c")
        s = lax.axis_index("s")
        z0 = s * ROWS_PER_TILE

        pltpu.sync_copy(zeros_hbm, zb)
        pltpu.sync_copy(ones_hbm, ob)

        def zero_body(j, carry):
            pltpu.sync_copy(zb, acc.at[pl.ds(z0 + j * CH, CH)])
            return carry
        lax.fori_loop(0, ROWS_PER_TILE // CH, zero_body, 0)
        plsc.subcore_barrier()

        base = (c * NS + s) * (CPT // NC)
        pltpu.sync_copy(dst_hbm.at[pl.ds(base, CPT // NC)], dst_v)

        def edge_body(j, carry):
            pltpu.sync_copy(ob, acc.at[dst_v.at[j]], add=True)
            return carry
        lax.fori_loop(0, CPT // NC, edge_body, 0)
        plsc.subcore_barrier()

        def out_body(j, carry):
            r0 = z0 + j * CH
            pltpu.sync_copy(acc.at[pl.ds(r0, CH)], zb)
            pltpu.sync_copy(zb, out_hbm.at[c, pl.ds(r0, CH)])
            return carry
        lax.fori_loop(0, ROWS_PER_TILE // CH, out_body, 0)

    return deg_kernel


def _make_agg_kernel(dh):
    mesh = plsc.VectorSubcoreMesh(core_axis_name="c", subcore_axis_name="s")

    @functools.partial(
        pl.kernel,
        out_type=jax.ShapeDtypeStruct((NC, R_ACC, dh), jnp.float32),
        mesh=mesh,
        scratch_types=(
            [pltpu.VMEM((PASS, CH), jnp.int32)] * 4
            + [pltpu.VMEM((CH, dh), jnp.float32)] * NBUF
            + [pltpu.VMEM_SHARED((N, dh), jnp.float32)]
            + [pltpu.VMEM_SHARED((R_ACC, dh), jnp.float32)]
            + [pltpu.SemaphoreType.DMA] * (NBUF + 2)
        ),
        compiler_params=pltpu.CompilerParams(use_tc_tiling_on_sc=False),
    )
    def agg_kernel(src_hbm, dst_hbm, hp_hbm, zeros_hbm, out_hbm, *rest):
        idx = rest[:4]
        gbs = rest[4:4 + NBUF]
        hp_sp = rest[4 + NBUF]
        acc = rest[5 + NBUF]
        gsems = rest[6 + NBUF:6 + 2 * NBUF]
        isems = rest[6 + 2 * NBUF:]
        c = lax.axis_index("c")
        s = lax.axis_index("s")
        z0 = s * ROWS_PER_TILE
        base = s * CPT

        @pl.when(s < CIN_TILES)
        def _():
            r = s * (N // CIN_TILES)
            pltpu.sync_copy(hp_hbm.at[c, pl.ds(r, N // CIN_TILES)],
                            hp_sp.at[pl.ds(r, N // CIN_TILES)])

        pltpu.sync_copy(zeros_hbm, gbs[0])

        def zero_body(j, carry):
            pltpu.sync_copy(gbs[0], acc.at[pl.ds(z0 + j * CH, CH)])
            return carry
        lax.fori_loop(0, ROWS_PER_TILE // CH, zero_body, 0)

        pltpu.sync_copy(src_hbm.at[pl.ds(base, PASS)], idx[0])
        pltpu.sync_copy(dst_hbm.at[pl.ds(base, PASS)], idx[1])
        plsc.subcore_barrier()
        for b in range(NBUF):
            pltpu.async_copy(hp_sp.at[idx[0].at[b]], gbs[b], gsems[b])

        for p in range(NPASS):
            sv, dv = idx[2 * (p % 2)], idx[2 * (p % 2) + 1]
            nsv, ndv = idx[2 * ((p + 1) % 2)], idx[2 * ((p + 1) % 2) + 1]
            if p + 1 < NPASS:
                nb = base + (p + 1) * PASS
                pltpu.async_copy(src_hbm.at[pl.ds(nb, PASS)], nsv, isems[0])
                pltpu.async_copy(dst_hbm.at[pl.ds(nb, PASS)], ndv, isems[1])
            if p > 0:
                for b in range(NBUF):
                    pltpu.async_copy(hp_sp.at[sv.at[b]], gbs[b], gsems[b])

            def grp_body(grp, carry):
                t0 = grp * NBUF
                for b in range(NBUF):
                    pltpu.make_async_copy(
                        hp_sp.at[sv.at[t0 + b]], gbs[b], gsems[b]).wait()
                    pltpu.sync_copy(gbs[b], acc.at[dv.at[t0 + b]], add=True)

                    @pl.when(grp + 1 < PGRP)
                    def _():
                        pltpu.async_copy(
                            hp_sp.at[sv.at[t0 + NBUF + b]], gbs[b], gsems[b])
                return carry
            lax.fori_loop(0, PGRP, grp_body, 0)

            if p + 1 < NPASS:
                nb = base + (p + 1) * PASS
                pltpu.make_async_copy(
                    src_hbm.at[pl.ds(nb, PASS)], nsv, isems[0]).wait()
                pltpu.make_async_copy(
                    dst_hbm.at[pl.ds(nb, PASS)], ndv, isems[1]).wait()
        plsc.subcore_barrier()

        def out_body(j, carry):
            r0 = z0 + j * CH
            pltpu.sync_copy(acc.at[pl.ds(r0, CH)], gbs[0])
            pltpu.sync_copy(gbs[0], out_hbm.at[c, pl.ds(r0, CH)])
            return carry
        lax.fori_loop(0, ROWS_PER_TILE // CH, out_body, 0)

    return agg_kernel


_deg_kernel = _make_deg_kernel()
_agg64 = _make_agg_kernel(H1 // 2)
_agg32 = _make_agg_kernel(H3 // 2)


def _mm_body(x_ref, w_ref, o_ref):
    o_ref[...] = jnp.dot(x_ref[...], w_ref[...],
                         preferred_element_type=jnp.float32)


def _mm(x, w):
    m, k = x.shape
    n = w.shape[1]
    return pl.pallas_call(
        _mm_body,
        grid=(GRID,),
        in_specs=[
            pl.BlockSpec((BLK, k), lambda i: (i, 0)),
            pl.BlockSpec((k, n), lambda i: (0, 0)),
        ],
        out_specs=pl.BlockSpec((BLK, n), lambda i: (i, 0)),
        out_shape=jax.ShapeDtypeStruct((m, n), jnp.float32),
    )(x, w)


def _scale_body(raw_ref, degp_ref, hp_ref, dis_ref):
    degp = degp_ref[...]
    deg = 1.0 + degp[0, :, 0:1] + degp[1, :, 0:1]
    dis = lax.rsqrt(deg)
    hp = raw_ref[...] * dis
    hp_ref[0] = hp[:, :H1 // 2]
    hp_ref[1] = hp[:, H1 // 2:]
    dis_ref[...] = jnp.broadcast_to(dis, (BLK, 16))


def _scale(raw, degp):
    return pl.pallas_call(
        _scale_body,
        grid=(GRID,),
        in_specs=[
            pl.BlockSpec((BLK, H1), lambda i: (i, 0)),
            pl.BlockSpec((NC, BLK, 16), lambda i: (0, i, 0)),
        ],
        out_specs=[
            pl.BlockSpec((NC, BLK, H1 // 2), lambda i: (0, i, 0)),
            pl.BlockSpec((BLK, 16), lambda i: (i, 0)),
        ],
        out_shape=[
            jax.ShapeDtypeStruct((NC, N, H1 // 2), jnp.float32),
            jax.ShapeDtypeStruct((N, 16), jnp.float32),
        ],
    )(raw, degp)


def _stats_body(p_ref, hp_ref, dis_ref, b_ref, t_ref, s_ref, ss_ref):
    p = p_ref[...]
    hp = hp_ref[...]
    agg = jnp.concatenate([p[0] + hp[0], p[1] + hp[1]], axis=1)
    t = agg * dis_ref[:, 0:1] + b_ref[...]
    t_ref[...] = t

    @pl.when(pl.program_id(0) == 0)
    def _():
        s_ref[...] = jnp.zeros_like(s_ref)
        ss_ref[...] = jnp.zeros_like(ss_ref)

    s_ref[...] += jnp.sum(t, axis=0, keepdims=True)
    ss_ref[...] += jnp.sum(t * t, axis=0, keepdims=True)


def _stats(p, hp, dis, b):
    dh = hp.shape[2]
    depth = 2 * dh
    return pl.pallas_call(
        _stats_body,
        grid=(GRID,),
        in_specs=[
            pl.BlockSpec((NC, BLK, dh), lambda i: (0, i, 0)),
            pl.BlockSpec((NC, BLK, dh), lambda i: (0, i, 0)),
            pl.BlockSpec((BLK, 16), lambda i: (i, 0)),
            pl.BlockSpec((1, depth), lambda i: (0, 0)),
        ],
        out_specs=[
            pl.BlockSpec((BLK, depth), lambda i: (i, 0)),
            pl.BlockSpec((1, depth), lambda i: (0, 0)),
            pl.BlockSpec((1, depth), lambda i: (0, 0)),
        ],
        out_shape=[
            jax.ShapeDtypeStruct((N, depth), jnp.float32),
            jax.ShapeDtypeStruct((1, depth), jnp.float32),
            jax.ShapeDtypeStruct((1, depth), jnp.float32),
        ],
    )(p, hp, dis, b)


def _bnmm_body(t_ref, s_ref, ss_ref, g_ref, bt_ref, w_ref, dis_ref, o_ref):
    m = s_ref[...] * (1.0 / N)
    v = ss_ref[...] * (1.0 / N) - m * m
    y = (t_ref[...] - m) * lax.rsqrt(v + 1e-5) * g_ref[...] + bt_ref[...]
    y = jnp.maximum(y, 0.0)
    o = jnp.dot(y, w_ref[...],
                preferred_element_type=jnp.float32) * dis_ref[:, 0:1]
    dn = o.shape[1]
    o_ref[0] = o[:, :dn // 2]
    o_ref[1] = o[:, dn // 2:]


def _bnmm(t, s, ss, g, bt, w, dis):
    depth = t.shape[1]
    dn = w.shape[1]
    return pl.pallas_call(
        _bnmm_body,
        grid=(GRID,),
        in_specs=[
            pl.BlockSpec((BLK, depth), lambda i: (i, 0)),
            pl.BlockSpec((1, depth), lambda i: (0, 0)),
            pl.BlockSpec((1, depth), lambda i: (0, 0)),
            pl.BlockSpec((1, depth), lambda i: (0, 0)),
            pl.BlockSpec((1, depth), lambda i: (0, 0)),
            pl.BlockSpec((depth, dn), lambda i: (0, 0)),
            pl.BlockSpec((BLK, 16), lambda i: (i, 0)),
        ],
        out_specs=pl.BlockSpec((NC, BLK, dn // 2), lambda i: (0, i, 0)),
        out_shape=jax.ShapeDtypeStruct((NC, N, dn // 2), jnp.float32),
    )(t, s, ss, g, bt, w, dis)


def _head_body(t_ref, s_ref, ss_ref, g_ref, bt_ref, cw1_ref, cb1_ref,
               cw2t_ref, cb2_ref, o_ref):
    m = s_ref[...] * (1.0 / N)
    v = ss_ref[...] * (1.0 / N) - m * m
    y = (t_ref[...] - m) * lax.rsqrt(v + 1e-5) * g_ref[...] + bt_ref[...]
    y = jnp.maximum(y, 0.0)
    h = jnp.dot(y, cw1_ref[...], preferred_element_type=jnp.float32)
    h = jnp.maximum(h + cb1_ref[...], 0.0)
    o_ref[...] = (jnp.sum(h * cw2t_ref[...], axis=1, keepdims=True)
                  + cb2_ref[...])


def _head(t, s, ss, g, bt, cw1, cb1, cw2t, cb2):
    return pl.pallas_call(
        _head_body,
        grid=(GRID,),
        in_specs=[
            pl.BlockSpec((BLK, H3), lambda i: (i, 0)),
            pl.BlockSpec((1, H3), lambda i: (0, 0)),
            pl.BlockSpec((1, H3), lambda i: (0, 0)),
            pl.BlockSpec((1, H3), lambda i: (0, 0)),
            pl.BlockSpec((1, H3), lambda i: (0, 0)),
            pl.BlockSpec((H3, 32), lambda i: (0, 0)),
            pl.BlockSpec((1, 32), lambda i: (0, 0)),
            pl.BlockSpec((1, 32), lambda i: (0, 0)),
            pl.BlockSpec((1, 1), lambda i: (0, 0)),
        ],
        out_specs=pl.BlockSpec((BLK, 1), lambda i: (i, 0)),
        out_shape=jax.ShapeDtypeStruct((N, 1), jnp.float32),
    )(t, s, ss, g, bt, cw1, cb1, cw2t, cb2)


@jax.jit
def _run(x, edge_index, W1, b1, g1, bt1, W2, b2, g2, bt2, W3, b3, g3, bt3,
         cW1, cb1, cW2, cb2):
    src = edge_index[0]
    dst = edge_index[1]
    pad = EP - E
    src_p = jnp.concatenate([src, jnp.zeros((pad,), jnp.int32)])
    dst_p = jnp.concatenate(
        [dst, N + (jnp.arange(pad, dtype=jnp.int32) % (R_ACC - N))])
    src2d = src_p.reshape(CHUNKS, CH)
    dst2d = dst_p.reshape(CHUNKS, CH)

    zeros16 = jnp.zeros((CH, 16), jnp.float32)
    ones16 = jnp.ones((CH, 16), jnp.float32)
    zeros64 = jnp.zeros((CH, H1 // 2), jnp.float32)
    zeros32 = jnp.zeros((CH, H3 // 2), jnp.float32)

    b1r = b1.reshape(1, H1)
    g1r = g1.reshape(1, H1)
    bt1r = bt1.reshape(1, H1)
    b2r = b2.reshape(1, H1)
    g2r = g2.reshape(1, H1)
    bt2r = bt2.reshape(1, H1)
    b3r = b3.reshape(1, H3)
    g3r = g3.reshape(1, H3)
    bt3r = bt3.reshape(1, H3)
    cb1r = cb1.reshape(1, 32)
    cw2t = cW2.reshape(1, 32)
    cb2r = cb2.reshape(1, 1)

    degp = _deg_kernel(dst2d, zeros16, ones16)
    raw1 = _mm(x, W1)
    hp1, dis = _scale(raw1, degp)

    p1 = _agg64(src2d, dst2d, hp1, zeros64)
    t1, s1, ss1 = _stats(p1, hp1, dis, b1r)
    hp2 = _bnmm(t1, s1, ss1, g1r, bt1r, W2, dis)

    p2 = _agg64(src2d, dst2d, hp2, zeros64)
    t2, s2, ss2 = _stats(p2, hp2, dis, b2r)
    hp3 = _bnmm(t2, s2, ss2, g2r, bt2r, W3, dis)

    p3 = _agg32(src2d, dst2d, hp3, zeros32)
    t3, s3, ss3 = _stats(p3, hp3, dis, b3r)
    out = _head(t3, s3, ss3, g3r, bt3r, cW1, cb1r, cw2t, cb2r)
    return out[:, 0]


def kernel(x, edge_index, W1, b1, g1, bt1, W2, b2, g2, bt2, W3, b3, g3, bt3,
           cW1, cb1, cW2, cb2):
    return _run(x, edge_index, W1, b1, g1, bt1, W2, b2, g2, bt2,
                W3, b3, g3, bt3, cW1, cb1, cW2, cb2)


# speedup vs baseline: 19.4849x; 1.0164x over previous
import functools

import jax
import jax.numpy as jnp
from jax import lax
from jax.experimental import pallas as pl
from jax.experimental.pallas import tpu as pltpu
from jax.experimental.pallas import tpu_sc as plsc

N = 10000
E = 320000
H1 = 128
H3 = 64

NC = 2
NS = 16
CH = 128

R_ACC = 10240
ROWS_PER_TILE = R_ACC // NS
EP = ((E + NS * CH * 8 - 1) // (NS * CH * 8)) * (NS * CH * 8)
CHUNKS = EP // CH
CPT = CHUNKS // NS

NBUF = 4
PASS = 16
NPASS = CPT // PASS
PGRP = PASS // NBUF

BLK = 1000
GRID = N // BLK
CIN_TILES = 10


def _make_deg_kernel():
    mesh = plsc.VectorSubcoreMesh(core_axis_name="c", subcore_axis_name="s")

    @functools.partial(
        pl.kernel,
        out_type=jax.ShapeDtypeStruct((NC, R_ACC, 16), jnp.float32),
        mesh=mesh,
        scratch_types=[
            pltpu.VMEM((CPT // NC, CH), jnp.int32),
            pltpu.VMEM((CH, 16), jnp.float32),
            pltpu.VMEM((CH, 16), jnp.float32),
            pltpu.VMEM_SHARED((R_ACC, 16), jnp.float32),
        ],
        compiler_params=pltpu.CompilerParams(use_tc_tiling_on_sc=False),
    )
    def deg_kernel(dst_hbm, zeros_hbm, ones_hbm, out_hbm, dst_v, zb, ob, acc):
        c = lax.axis_index("c")
        s = lax.axis_index("s")
        z0 = s * ROWS_PER_TILE

        pltpu.sync_copy(zeros_hbm, zb)
        pltpu.sync_copy(ones_hbm, ob)

        def zero_body(j, carry):
            pltpu.sync_copy(zb, acc.at[pl.ds(z0 + j * CH, CH)])
            return carry
        lax.fori_loop(0, ROWS_PER_TILE // CH, zero_body, 0)
        plsc.subcore_barrier()

        base = (c * NS + s) * (CPT // NC)
        pltpu.sync_copy(dst_hbm.at[pl.ds(base, CPT // NC)], dst_v)

        def edge_body(j, carry):
            pltpu.sync_copy(ob, acc.at[dst_v.at[j]], add=True)
            return carry
        lax.fori_loop(0, CPT // NC, edge_body, 0)
        plsc.subcore_barrier()

        def out_body(j, carry):
            r0 = z0 + j * CH
            pltpu.sync_copy(acc.at[pl.ds(r0, CH)], zb)
            pltpu.sync_copy(zb, out_hbm.at[c, pl.ds(r0, CH)])
            return carry
        lax.fori_loop(0, ROWS_PER_TILE // CH, out_body, 0)

    return deg_kernel


def _make_agg_kernel(dh):
    mesh = plsc.VectorSubcoreMesh(core_axis_name="c", subcore_axis_name="s")

    @functools.partial(
        pl.kernel,
        out_type=jax.ShapeDtypeStruct((NC, R_ACC, dh), jnp.float32),
        mesh=mesh,
        scratch_types=(
            [pltpu.VMEM((PASS, CH), jnp.int32)] * 4
            + [pltpu.VMEM((CH, dh), jnp.float32)] * NBUF
            + [pltpu.VMEM_SHARED((N, dh), jnp.float32)]
            + [pltpu.VMEM_SHARED((R_ACC, dh), jnp.float32)]
            + [pltpu.SemaphoreType.DMA] * (NBUF + 2)
        ),
        compiler_params=pltpu.CompilerParams(use_tc_tiling_on_sc=False),
    )
    def agg_kernel(src_hbm, dst_hbm, hp_hbm, zeros_hbm, out_hbm, *rest):
        idx = rest[:4]
        gbs = rest[4:4 + NBUF]
        hp_sp = rest[4 + NBUF]
        acc = rest[5 + NBUF]
        gsems = rest[6 + NBUF:6 + 2 * NBUF]
        isems = rest[6 + 2 * NBUF:]
        c = lax.axis_index("c")
        s = lax.axis_index("s")
        z0 = s * ROWS_PER_TILE
        base = s * CPT

        @pl.when(s < CIN_TILES)
        def _():
            r = s * (N // CIN_TILES)
            pltpu.sync_copy(hp_hbm.at[c, pl.ds(r, N // CIN_TILES)],
                            hp_sp.at[pl.ds(r, N // CIN_TILES)])

        pltpu.sync_copy(zeros_hbm, gbs[0])

        def zero_body(j, carry):
            pltpu.sync_copy(gbs[0], acc.at[pl.ds(z0 + j * CH, CH)])
            return carry
        lax.fori_loop(0, ROWS_PER_TILE // CH, zero_body, 0)

        pltpu.sync_copy(src_hbm.at[pl.ds(base, PASS)], idx[0])
        pltpu.sync_copy(dst_hbm.at[pl.ds(base, PASS)], idx[1])
        plsc.subcore_barrier()
        for b in range(NBUF):
            pltpu.async_copy(hp_sp.at[idx[0].at[b]], gbs[b], gsems[b])

        for p in range(NPASS):
            sv, dv = idx[2 * (p % 2)], idx[2 * (p % 2) + 1]
            nsv, ndv = idx[2 * ((p + 1) % 2)], idx[2 * ((p + 1) % 2) + 1]
            if p + 1 < NPASS:
                nb = base + (p + 1) * PASS
                pltpu.async_copy(src_hbm.at[pl.ds(nb, PASS)], nsv, isems[0])
                pltpu.async_copy(dst_hbm.at[pl.ds(nb, PASS)], ndv, isems[1])
            if p > 0:
                for b in range(NBUF):
                    pltpu.async_copy(hp_sp.at[sv.at[b]], gbs[b], gsems[b])

            def grp_body(grp, carry):
                t0 = grp * NBUF
                for b in range(NBUF):
                    pltpu.make_async_copy(
                        hp_sp.at[sv.at[t0 + b]], gbs[b], gsems[b]).wait()
                    pltpu.sync_copy(gbs[b], acc.at[dv.at[t0 + b]], add=True)

                    @pl.when(grp + 1 < PGRP)
                    def _():
                        pltpu.async_copy(
                            hp_sp.at[sv.at[t0 + NBUF + b]], gbs[b], gsems[b])
                return carry
            lax.fori_loop(0, PGRP, grp_body, 0)

            if p + 1 < NPASS:
                nb = base + (p + 1) * PASS
                pltpu.make_async_copy(
                    src_hbm.at[pl.ds(nb, PASS)], nsv, isems[0]).wait()
                pltpu.make_async_copy(
                    dst_hbm.at[pl.ds(nb, PASS)], ndv, isems[1]).wait()
        plsc.subcore_barrier()

        def out_body(j, carry):
            r0 = z0 + j * CH
            pltpu.sync_copy(acc.at[pl.ds(r0, CH)], gbs[0])
            pltpu.sync_copy(gbs[0], out_hbm.at[c, pl.ds(r0, CH)])
            return carry
        lax.fori_loop(0, ROWS_PER_TILE // CH, out_body, 0)

    return agg_kernel


_deg_kernel = _make_deg_kernel()
_agg64 = _make_agg_kernel(H1 // 2)
_agg32 = _make_agg_kernel(H3 // 2)


def _mm_body(x_ref, w_ref, o_ref):
    o_ref[...] = jnp.dot(x_ref[...], w_ref[...],
                         preferred_element_type=jnp.float32)


def _mm(x, w):
    m, k = x.shape
    n = w.shape[1]
    return pl.pallas_call(
        _mm_body,
        grid=(GRID,),
        in_specs=[
            pl.BlockSpec((BLK, k), lambda i: (i, 0)),
            pl.BlockSpec((k, n), lambda i: (0, 0)),
        ],
        out_specs=pl.BlockSpec((BLK, n), lambda i: (i, 0)),
        out_shape=jax.ShapeDtypeStruct((m, n), jnp.float32),
    )(x, w)


def _scale_body(raw_ref, degp_ref, hp_ref, dis_ref):
    degp = degp_ref[...]
    deg = 1.0 + degp[0, :, 0:1] + degp[1, :, 0:1]
    dis = lax.rsqrt(deg)
    hp = raw_ref[...] * dis
    hp_ref[0] = hp[:, :H1 // 2]
    hp_ref[1] = hp[:, H1 // 2:]
    dis_ref[...] = jnp.broadcast_to(dis, (BLK, 16))


def _scale(raw, degp):
    return pl.pallas_call(
        _scale_body,
        grid=(GRID,),
        in_specs=[
            pl.BlockSpec((BLK, H1), lambda i: (i, 0)),
            pl.BlockSpec((NC, BLK, 16), lambda i: (0, i, 0)),
        ],
        out_specs=[
            pl.BlockSpec((NC, BLK, H1 // 2), lambda i: (0, i, 0)),
            pl.BlockSpec((BLK, 16), lambda i: (i, 0)),
        ],
        out_shape=[
            jax.ShapeDtypeStruct((NC, N, H1 // 2), jnp.float32),
            jax.ShapeDtypeStruct((N, 16), jnp.float32),
        ],
    )(raw, degp)


def _bn_stats(t_sc, s_sc, ss_sc, i, t):
    t_sc[pl.ds(i * BLK, BLK), :] = t

    @pl.when(i == 0)
    def _():
        s_sc[...] = jnp.zeros_like(s_sc)
        ss_sc[...] = jnp.zeros_like(ss_sc)

    s_sc[...] += jnp.sum(t, axis=0, keepdims=True)
    ss_sc[...] += jnp.sum(t * t, axis=0, keepdims=True)


def _bn_apply(t_sc, s_sc, ss_sc, j, g, bt):
    t = t_sc[pl.ds(j * BLK, BLK), :]
    m = s_sc[...] * (1.0 / N)
    v = ss_sc[...] * (1.0 / N) - m * m
    return jnp.maximum((t - m) * lax.rsqrt(v + 1e-5) * g + bt, 0.0)


def _layer_body(p_ref, hp_ref, dis_ref, b_ref, g_ref, bt_ref, w_ref,
                o_ref, t_sc, s_sc, ss_sc):
    i = pl.program_id(0)

    @pl.when(i < GRID)
    def _():
        p = p_ref[...]
        hp = hp_ref[...]
        agg = jnp.concatenate([p[0] + hp[0], p[1] + hp[1]], axis=1)
        t = agg * dis_ref[:, 0:1] + b_ref[...]
        _bn_stats(t_sc, s_sc, ss_sc, i, t)

    @pl.when(i >= GRID)
    def _():
        y = _bn_apply(t_sc, s_sc, ss_sc, i - GRID, g_ref[...], bt_ref[...])
        o = jnp.dot(y, w_ref[...],
                    preferred_element_type=jnp.float32) * dis_ref[:, 0:1]
        dn = o.shape[1]
        o_ref[0] = o[:, :dn // 2]
        o_ref[1] = o[:, dn // 2:]


def _layer(p, hp, dis, b, g, bt, w):
    dh = hp.shape[2]
    depth = 2 * dh
    dn = w.shape[1]

    def park(i):
        return jnp.minimum(i, GRID - 1)

    def wrap(i):
        return jnp.where(i < GRID, i, i - GRID)

    return pl.pallas_call(
        _layer_body,
        grid=(2 * GRID,),
        in_specs=[
            pl.BlockSpec((NC, BLK, dh), lambda i: (0, park(i), 0)),
            pl.BlockSpec((NC, BLK, dh), lambda i: (0, park(i), 0)),
            pl.BlockSpec((BLK, 16), lambda i: (wrap(i), 0)),
            pl.BlockSpec((1, depth), lambda i: (0, 0)),
            pl.BlockSpec((1, depth), lambda i: (0, 0)),
            pl.BlockSpec((1, depth), lambda i: (0, 0)),
            pl.BlockSpec((depth, dn), lambda i: (0, 0)),
        ],
        out_specs=pl.BlockSpec(
            (NC, BLK, dn // 2),
            lambda i: (0, jnp.where(i < GRID, 0, i - GRID), 0)),
        out_shape=jax.ShapeDtypeStruct((NC, N, dn // 2), jnp.float32),
        scratch_shapes=[
            pltpu.VMEM((N, depth), jnp.float32),
            pltpu.VMEM((1, depth), jnp.float32),
            pltpu.VMEM((1, depth), jnp.float32),
        ],
    )(p, hp, dis, b, g, bt, w)


def _head_body(p_ref, hp_ref, dis_ref, b_ref, g_ref, bt_ref, cw1_ref,
               cb1_ref, cw2t_ref, cb2_ref, o_ref, t_sc, s_sc, ss_sc):
    i = pl.program_id(0)

    @pl.when(i < GRID)
    def _():
        p = p_ref[...]
        hp = hp_ref[...]
        agg = jnp.concatenate([p[0] + hp[0], p[1] + hp[1]], axis=1)
        t = agg * dis_ref[:, 0:1] + b_ref[...]
        _bn_stats(t_sc, s_sc, ss_sc, i, t)

    @pl.when(i >= GRID)
    def _():
        y = _bn_apply(t_sc, s_sc, ss_sc, i - GRID, g_ref[...], bt_ref[...])
        h = jnp.dot(y, cw1_ref[...], preferred_element_type=jnp.float32)
        h = jnp.maximum(h + cb1_ref[...], 0.0)
        o_ref[...] = (jnp.sum(h * cw2t_ref[...], axis=1, keepdims=True)
                      + cb2_ref[...])


def _head(p, hp, dis, b, g, bt, cw1, cb1, cw2t, cb2):
    dh = hp.shape[2]
    depth = 2 * dh

    def park(i):
        return jnp.minimum(i, GRID - 1)

    def wrap(i):
        return jnp.where(i < GRID, i, i - GRID)

    return pl.pallas_call(
        _head_body,
        grid=(2 * GRID,),
        in_specs=[
            pl.BlockSpec((NC, BLK, dh), lambda i: (0, park(i), 0)),
            pl.BlockSpec((NC, BLK, dh), lambda i: (0, park(i), 0)),
            pl.BlockSpec((BLK, 16), lambda i: (wrap(i), 0)),
            pl.BlockSpec((1, depth), lambda i: (0, 0)),
            pl.BlockSpec((1, depth), lambda i: (0, 0)),
            pl.BlockSpec((1, depth), lambda i: (0, 0)),
            pl.BlockSpec((H3, 32), lambda i: (0, 0)),
            pl.BlockSpec((1, 32), lambda i: (0, 0)),
            pl.BlockSpec((1, 32), lambda i: (0, 0)),
            pl.BlockSpec((1, 1), lambda i: (0, 0)),
        ],
        out_specs=pl.BlockSpec(
            (BLK, 1), lambda i: (jnp.where(i < GRID, 0, i - GRID), 0)),
        out_shape=jax.ShapeDtypeStruct((N, 1), jnp.float32),
        scratch_shapes=[
            pltpu.VMEM((N, depth), jnp.float32),
            pltpu.VMEM((1, depth), jnp.float32),
            pltpu.VMEM((1, depth), jnp.float32),
        ],
    )(p, hp, dis, b, g, bt, cw1, cb1, cw2t, cb2)


@jax.jit
def _run(x, edge_index, W1, b1, g1, bt1, W2, b2, g2, bt2, W3, b3, g3, bt3,
         cW1, cb1, cW2, cb2):
    src = edge_index[0]
    dst = edge_index[1]
    pad = EP - E
    src_p = jnp.concatenate([src, jnp.zeros((pad,), jnp.int32)])
    dst_p = jnp.concatenate(
        [dst, N + (jnp.arange(pad, dtype=jnp.int32) % (R_ACC - N))])
    src2d = src_p.reshape(CHUNKS, CH)
    dst2d = dst_p.reshape(CHUNKS, CH)

    zeros16 = jnp.zeros((CH, 16), jnp.float32)
    ones16 = jnp.ones((CH, 16), jnp.float32)
    zeros64 = jnp.zeros((CH, H1 // 2), jnp.float32)
    zeros32 = jnp.zeros((CH, H3 // 2), jnp.float32)

    b1r = b1.reshape(1, H1)
    g1r = g1.reshape(1, H1)
    bt1r = bt1.reshape(1, H1)
    b2r = b2.reshape(1, H1)
    g2r = g2.reshape(1, H1)
    bt2r = bt2.reshape(1, H1)
    b3r = b3.reshape(1, H3)
    g3r = g3.reshape(1, H3)
    bt3r = bt3.reshape(1, H3)
    cb1r = cb1.reshape(1, 32)
    cw2t = cW2.reshape(1, 32)
    cb2r = cb2.reshape(1, 1)

    degp = _deg_kernel(dst2d, zeros16, ones16)
    raw1 = _mm(x, W1)
    hp1, dis = _scale(raw1, degp)

    p1 = _agg64(src2d, dst2d, hp1, zeros64)
    hp2 = _layer(p1, hp1, dis, b1r, g1r, bt1r, W2)

    p2 = _agg64(src2d, dst2d, hp2, zeros64)
    hp3 = _layer(p2, hp2, dis, b2r, g2r, bt2r, W3)

    p3 = _agg32(src2d, dst2d, hp3, zeros32)
    out = _head(p3, hp3, dis, b3r, g3r, bt3r, cW1, cb1r, cw2t, cb2r)
    return out[:, 0]


def kernel(x, edge_index, W1, b1, g1, bt1, W2, b2, g2, bt2, W3, b3, g3, bt3,
           cW1, cb1, cW2, cb2):
    return _run(x, edge_index, W1, b1, g1, bt1, W2, b2, g2, bt2,
                W3, b3, g3, bt3, cW1, cb1, cW2, cb2)
